# Initial kernel scaffold; baseline (speedup 1.0000x reference)
import jax
import jax.numpy as jnp
from jax.experimental import pallas as pl


def kernel(robot_positions, robot_activities, consciousness_field, resonance_field, genetic_field):
    raise NotImplementedError("write your pallas kernel here")


# speedup vs baseline: 2423.3831x; 2423.3831x over previous
import functools

import jax
import jax.numpy as jnp
from jax import lax
from jax.experimental import pallas as pl
from jax.experimental.pallas import tpu as pltpu
from jax.experimental.pallas import tpu_sc as plsc

_G = 256
_W = 100.0
_DIFF = 0.1
_DECAY = 0.01
_DT = 0.1
_N = 1024
_NC = 2
_NS = 16
_NW = _NC * _NS
_PAIRS = _N * _N
_PER_W = _PAIRS // _NW
_CH = 128
_NCHUNK = _PER_W // _CH
_CELLS = _G * _G


def _pairs_body(pxc_ref, pyc_ref, pxr_ref, pyr_ref, ra_ref, w_ref, cell_ref):
    pxc = pxc_ref[...]
    pyc = pyc_ref[...]
    pxr = pxr_ref[...]
    pyr = pyr_ref[...]
    ra = ra_ref[...]

    nrm = jnp.sqrt(jnp.sum(ra * ra, axis=1, keepdims=True))
    an16 = ra[:, :16] / (nrm + 1e-8)
    sim = lax.dot_general(an16, an16, (((1,), (1,)), ((), ())),
                          preferred_element_type=jnp.float32)

    dx = pxc - pxr
    dy = pyc - pyr
    dist = jnp.sqrt(dx * dx + dy * dy)
    near = (dist < 10.0).astype(jnp.float32)

    ii = lax.broadcasted_iota(jnp.int32, (_N, _N), 0)
    jj = lax.broadcasted_iota(jnp.int32, (_N, _N), 1)
    offdiag = (ii != jj).astype(jnp.float32)

    sig = 1.0 / (1.0 + jnp.exp(-5.0 * sim))
    w_ref[...] = sig * near * offdiag * (0.5 * _DT)

    midx = (pxc + pxr) * 0.5
    midy = (pyc + pyr) * 0.5
    cx = jnp.clip((midx / _W * _G).astype(jnp.int32), 0, _G - 1)
    cy = jnp.clip((midy / _W * _G).astype(jnp.int32), 0, _G - 1)
    cell_ref[...] = cx * _G + cy


def _pairs_call(pxc, pyc, pxr, pyr, ra):
    return pl.pallas_call(
        _pairs_body,
        out_shape=(
            jax.ShapeDtypeStruct((_N, _N), jnp.float32),
            jax.ShapeDtypeStruct((_N, _N), jnp.int32),
        ),
    )(pxc, pyc, pxr, pyr, ra)


def _sc_scatter_body(w_hbm, idx_hbm, out_hbm, vals_v, idx_v, zero_v, acc_sh):
    c = lax.axis_index("c")
    s = lax.axis_index("s")
    wid = s * _NC + c
    sslice = pl.ds(s * (_CELLS // _NS), _CELLS // _NS)

    def _zbody(i, carry):
        zero_v[pl.ds(i * 16, 16)] = jnp.zeros((16,), jnp.float32)
        return carry

    lax.fori_loop(0, (_CELLS // _NS) // 16, _zbody, 0)
    pltpu.sync_copy(zero_v, acc_sh.at[sslice])
    plsc.subcore_barrier()

    row0 = wid * _NCHUNK
    pltpu.sync_copy(w_hbm.at[pl.ds(row0, _NCHUNK)], vals_v)
    pltpu.sync_copy(idx_hbm.at[pl.ds(row0, _NCHUNK)], idx_v)

    def _sbody(t, carry):
        pltpu.sync_copy(vals_v.at[t], acc_sh.at[idx_v.at[t]], add=True)
        return carry

    lax.fori_loop(0, _NCHUNK, _sbody, 0)
    plsc.subcore_barrier()
    pltpu.sync_copy(acc_sh.at[sslice], out_hbm.at[c, sslice])


def _sc_scatter(w_flat, cell_flat):
    mesh = plsc.VectorSubcoreMesh(core_axis_name="c", subcore_axis_name="s")
    fn = functools.partial(
        pl.kernel,
        out_type=jax.ShapeDtypeStruct((_NC, _CELLS), jnp.float32),
        mesh=mesh,
        scratch_types=[
            pltpu.VMEM((_NCHUNK, _CH), jnp.float32),
            pltpu.VMEM((_NCHUNK, _CH), jnp.int32),
            pltpu.VMEM((_CELLS // _NS,), jnp.float32),
            pltpu.VMEM_SHARED((_CELLS,), jnp.float32),
        ],
    )(_sc_scatter_body)
    return fn(w_flat, cell_flat)


_WB = 32
_NB = _G // _WB


def _scatter_body(pxc_ref, pyc_ref, ra_ref, cf_ref, gf_ref,
                  cfp_ref, gfo_ref):
    wb = pl.program_id(0)
    pxc = pxc_ref[...]
    pyc = pyc_ref[...]
    ra = ra_ref[...]

    gxq = jnp.clip((pxc / _W * _G).astype(jnp.int32), 0, _G - 1)
    gyq = jnp.clip((pyc / _W * _G).astype(jnp.int32), 0, _G - 1)

    hrow = lax.broadcasted_iota(jnp.int32, (1, _G), 1)
    hx = (gxq == hrow).astype(jnp.float32)

    j16 = lax.broadcasted_iota(jnp.int32, (1, _WB * 16), 1)
    hy16 = (gyq == (wb * _WB + j16 // 16)).astype(jnp.float32)
    c16 = lax.broadcasted_iota(jnp.int32, (16, 1), 0)
    l16 = (c16 == (j16 % 16)).astype(jnp.float32)
    a16 = ra[:, :16] * _DT
    a16t = lax.dot_general(a16, l16, (((1,), (0,)), ((), ())),
                           preferred_element_type=jnp.float32)
    m16 = hy16 * a16t
    upd16 = lax.dot_general(hx, m16, (((0,), (0,)), ((), ())),
                            preferred_element_type=jnp.float32)
    cfp_ref[...] = cf_ref[...] + upd16

    j32 = lax.broadcasted_iota(jnp.int32, (1, _WB * 32), 1)
    hy32 = (gyq == (wb * _WB + j32 // 32)).astype(jnp.float32)
    c32 = lax.broadcasted_iota(jnp.int32, (32, 1), 0)
    l32 = (c32 == (j32 % 32)).astype(jnp.float32)
    a32 = ra[:, :32] * (_DT * 0.1)
    a32t = lax.dot_general(a32, l32, (((1,), (0,)), ((), ())),
                           preferred_element_type=jnp.float32)
    m32 = hy32 * a32t
    upd32 = lax.dot_general(hx, m32, (((0,), (0,)), ((), ())),
                            preferred_element_type=jnp.float32)
    gfo_ref[...] = (gf_ref[...] + upd32) * (1.0 - _DECAY * 0.1 * _DT)


def _scatter_call(pxc, pyc, ra, cfr, gfr):
    n16 = _WB * 16
    n32 = _WB * 32
    return pl.pallas_call(
        _scatter_body,
        grid=(_NB,),
        in_specs=[
            pl.BlockSpec((_N, 1), lambda wb: (0, 0)),
            pl.BlockSpec((_N, 1), lambda wb: (0, 0)),
            pl.BlockSpec((_N, 64), lambda wb: (0, 0)),
            pl.BlockSpec((_G, n16), lambda wb: (0, wb)),
            pl.BlockSpec((_G, n32), lambda wb: (0, wb)),
        ],
        out_specs=[
            pl.BlockSpec((_G, n16), lambda wb: (0, wb)),
            pl.BlockSpec((_G, n32), lambda wb: (0, wb)),
        ],
        out_shape=[
            jax.ShapeDtypeStruct((_G, _G * 16), jnp.float32),
            jax.ShapeDtypeStruct((_G, _G * 32), jnp.float32),
        ],
    )(pxc, pyc, ra, cfr, gfr)


def _lap(x, ch):
    up = jnp.concatenate([x[:1, :], x[:-1, :]], axis=0)
    dn = jnp.concatenate([x[1:, :], x[-1:, :]], axis=0)
    lf = jnp.concatenate([x[:, :ch], x[:, :-ch]], axis=1)
    rt = jnp.concatenate([x[:, ch:], x[:, -ch:]], axis=1)
    return up + dn + lf + rt - 4.0 * x


def _diffuse_body(cfp_ref, rf_ref, r2_ref, cfo_ref, rfo_ref):
    cf = cfp_ref[...]
    rsum = r2_ref[0] + r2_ref[1]
    wcol = lax.broadcasted_iota(jnp.int32, (_G, 1), 0)
    j16 = lax.broadcasted_iota(jnp.int32, (1, _G * 16), 1)
    k16 = (wcol == (j16 // 16)).astype(jnp.float32)
    rexp = lax.dot_general(rsum, k16, (((1,), (0,)), ((), ())),
                           preferred_element_type=jnp.float32)
    rf = rf_ref[...] + rexp
    cfo_ref[...] = (cf + (_DIFF * _DT) * _lap(cf, 16)) * (1.0 - _DECAY * _DT)
    rfo_ref[...] = (rf + (_DIFF * 2.0 * _DT) * _lap(rf, 16)) * (1.0 - _DECAY * 0.5 * _DT)


def _diffuse_call(cfp, rfr, r2g):
    return pl.pallas_call(
        _diffuse_body,
        out_shape=(
            jax.ShapeDtypeStruct((_G, _G * 16), jnp.float32),
            jax.ShapeDtypeStruct((_G, _G * 16), jnp.float32),
        ),
    )(cfp, rfr, r2g)


def kernel(robot_positions, robot_activities, consciousness_field,
           resonance_field, genetic_field):
    pxc = robot_positions[:, 0:1]
    pyc = robot_positions[:, 1:2]
    pxr = jnp.reshape(pxc, (1, _N))
    pyr = jnp.reshape(pyc, (1, _N))

    w, cell = _pairs_call(pxc, pyc, pxr, pyr, robot_activities)
    r2 = _sc_scatter(jnp.reshape(w, (_PAIRS // _CH, _CH)),
                     jnp.reshape(cell, (_PAIRS // _CH, _CH)))

    cfr = jnp.reshape(consciousness_field, (_G, _G * 16))
    rfr = jnp.reshape(resonance_field, (_G, _G * 16))
    gfr = jnp.reshape(genetic_field, (_G, _G * 32))
    r2g = jnp.reshape(r2, (_NC, _G, _G))

    cfp, gfo = _scatter_call(pxc, pyc, robot_activities, cfr, gfr)
    cfo, rfo = _diffuse_call(cfp, rfr, r2g)

    return (jnp.reshape(cfo, (_G, _G, 16)),
            jnp.reshape(rfo, (_G, _G, 16)),
            jnp.reshape(gfo, (_G, _G, 32)))
